# Initial kernel scaffold; baseline (speedup 1.0000x reference)
import jax
import jax.numpy as jnp
from jax.experimental import pallas as pl


def kernel(x_encounter, x_patient, edge_index_enc_to_pat, edge_index_pat_to_enc, W_enc, b_enc, emb_pat, W_rel, W_root, b_conv, W_cls, b_cls):
    raise NotImplementedError("write your pallas kernel here")


# speedup vs baseline: 1.1557x; 1.1557x over previous
import functools

import jax
import jax.numpy as jnp
from jax.experimental import pallas as pl
from jax.experimental.pallas import tpu as pltpu

_NE = 50000
_NP = 50000
_E = 300000
_H = 128


def _mm_bias(x, W, b, block=1000):
    N, H = x.shape
    Ho = W.shape[1]
    b2 = b.reshape(1, Ho)

    def body(x_ref, w_ref, b_ref, o_ref):
        o_ref[...] = jnp.dot(x_ref[...], w_ref[...],
                             preferred_element_type=jnp.float32) + b_ref[...]

    return pl.pallas_call(
        body,
        grid=(N // block,),
        in_specs=[pl.BlockSpec((block, H), lambda i: (i, 0)),
                  pl.BlockSpec((H, Ho), lambda i: (0, 0)),
                  pl.BlockSpec((1, Ho), lambda i: (0, 0))],
        out_specs=pl.BlockSpec((block, Ho), lambda i: (i, 0)),
        out_shape=jax.ShapeDtypeStruct((N, Ho), jnp.float32),
    )(x, W, b2)


def _mm(x, W, block=1000):
    N, H = x.shape
    Ho = W.shape[1]

    def body(x_ref, w_ref, o_ref):
        o_ref[...] = jnp.dot(x_ref[...], w_ref[...],
                             preferred_element_type=jnp.float32)

    return pl.pallas_call(
        body,
        grid=(N // block,),
        in_specs=[pl.BlockSpec((block, H), lambda i: (i, 0)),
                  pl.BlockSpec((H, Ho), lambda i: (0, 0))],
        out_specs=pl.BlockSpec((block, Ho), lambda i: (i, 0)),
        out_shape=jax.ShapeDtypeStruct((N, Ho), jnp.float32),
    )(x, W)


def _seg_mean(h, src, dst, n_out, inv):
    m = jnp.take(h, src, axis=0)
    s = jax.ops.segment_sum(m, dst, num_segments=n_out)
    return s * inv[:, None]


def kernel(x_encounter, x_patient, edge_index_enc_to_pat, edge_index_pat_to_enc,
           W_enc, b_enc, emb_pat, W_rel, W_root, b_conv, W_cls, b_cls):
    f32 = jnp.float32
    src0 = edge_index_enc_to_pat[0]
    dst0 = edge_index_enc_to_pat[1]
    src1 = edge_index_pat_to_enc[0]
    dst1 = edge_index_pat_to_enc[1]

    ones = jnp.ones((_E,), f32)
    c0 = jax.ops.segment_sum(ones, dst0, num_segments=_NP)
    c1 = jax.ops.segment_sum(ones, dst1, num_segments=_NE)
    inv0 = 1.0 / jnp.maximum(c0, 1.0)
    inv1 = 1.0 / jnp.maximum(c1, 1.0)

    enc = _mm_bias(x_encounter, W_enc, b_enc)
    pat = jnp.take(emb_pat, x_patient, axis=0)

    A_enc = _mm_bias(enc, W_root[0], b_conv[0])
    A_pat = _mm_bias(pat, W_root[0], b_conv[0])
    h0 = _mm(enc, W_rel[0, 0])
    h1 = _mm(pat, W_rel[0, 1])
    s0 = _seg_mean(h0, src0, dst0, _NP, inv0)
    s1 = _seg_mean(h1, src1, dst1, _NE, inv1)
    enc = jax.nn.relu(A_enc + s1)
    pat = jax.nn.relu(A_pat + s0)

    A_enc = _mm_bias(enc, W_root[1], b_conv[1])
    h1 = _mm(pat, W_rel[1, 1])
    s1 = _seg_mean(h1, src1, dst1, _NE, inv1)
    enc = jax.nn.relu(A_enc + s1)

    logit = (enc @ W_cls + b_cls).reshape(-1)
    return logit


# speedup vs baseline: 3.5218x; 3.0474x over previous
import functools

import jax
import jax.numpy as jnp
from jax import lax
from jax.experimental import pallas as pl
from jax.experimental.pallas import tpu as pltpu
from jax.experimental.pallas import tpu_sc as plsc

_NE = 50000
_NP = 50000
_E = 300000
_H = 128

_NWORK = 16
_WIN = 128
_WPW = 148
_EPW = _WIN * _WPW
_EPAD = _EPW * _NWORK
_SENT = 176
_NROWS = _NP + _SENT
_ZCH = _NROWS // _NWORK
_FCH = _ZCH
_FLAST = _NP - 15 * _FCH


def _mm_bias(x, W, b, block=1000):
    N, H = x.shape
    Ho = W.shape[1]
    b2 = b.reshape(1, Ho)

    def body(x_ref, w_ref, b_ref, o_ref):
        o_ref[...] = jnp.dot(x_ref[...], w_ref[...],
                             preferred_element_type=jnp.float32) + b_ref[...]

    return pl.pallas_call(
        body,
        grid=(N // block,),
        in_specs=[pl.BlockSpec((block, H), lambda i: (i, 0)),
                  pl.BlockSpec((H, Ho), lambda i: (0, 0)),
                  pl.BlockSpec((1, Ho), lambda i: (0, 0))],
        out_specs=pl.BlockSpec((block, Ho), lambda i: (i, 0)),
        out_shape=jax.ShapeDtypeStruct((N, Ho), jnp.float32),
    )(x, W, b2)


def _mm_slices(x, W, block=1000):
    N, H = x.shape

    def body(x_ref, w_ref, *outs):
        r = jnp.dot(x_ref[...], w_ref[...], preferred_element_type=jnp.float32)
        for q, o in enumerate(outs):
            o[...] = r[:, q * 32:(q + 1) * 32]

    return pl.pallas_call(
        body,
        grid=(N // block,),
        in_specs=[pl.BlockSpec((block, H), lambda i: (i, 0)),
                  pl.BlockSpec((H, H), lambda i: (0, 0))],
        out_specs=[pl.BlockSpec((block, 32), lambda i: (i, 0))] * 4,
        out_shape=[jax.ShapeDtypeStruct((N, 32), jnp.float32)] * 4,
    )(x, W)


def _pad_edges(src, dst):
    pad = _EPAD - _E
    i = jnp.arange(pad, dtype=jnp.int32)
    src_p = jnp.concatenate([src.astype(jnp.int32), i % _NE])
    dst_p = jnp.concatenate([dst.astype(jnp.int32), _NP + (i % _SENT)])
    return (src_p.reshape(_NWORK, _WPW, _WIN),
            dst_p.reshape(_NWORK, _WPW, _WIN))


def _seg_sum_sc(hq, src3, dst3):
    mesh = plsc.VectorSubcoreMesh(core_axis_name="c", subcore_axis_name="s")

    @functools.partial(
        pl.kernel, mesh=mesh,
        compiler_params=pltpu.CompilerParams(use_tc_tiling_on_sc=False),
        out_type=[jax.ShapeDtypeStruct((_NP, 32), jnp.float32)] * 4,
        scratch_types=[
            pltpu.VMEM((2, _WIN), jnp.int32),
            pltpu.VMEM((2, _WIN), jnp.int32),
            pltpu.VMEM((2, _WIN, 32), jnp.float32),
            pltpu.VMEM((196, 32), jnp.float32),
            pltpu.VMEM_SHARED((_NROWS, 32), jnp.float32),
            pltpu.SemaphoreType.DMA,
            pltpu.SemaphoreType.DMA,
        ],
    )
    def k(h0, h1, h2, h3, src_h, dst_h, o0, o1, o2, o3,
          src_v, dst_v, rows_v, zero_v, acc, sem0, sem1):
        c = lax.axis_index("c")
        s = lax.axis_index("s")

        def zrow(i, carry):
            zero_v[i, 0:16] = jnp.zeros((16,), jnp.float32)
            zero_v[i, 16:32] = jnp.zeros((16,), jnp.float32)
            return carry

        lax.fori_loop(0, 196, zrow, 0)
        sems = (sem0, sem1)

        def run_quarter(tab, out):
            def zchunk(i, carry):
                pltpu.sync_copy(zero_v,
                                acc.at[pl.ds(s * _ZCH + i * 196, 196)])
                return carry

            lax.fori_loop(0, _ZCH // 196, zchunk, 0)
            plsc.subcore_barrier()

            def fetch_idx(j, b):
                pltpu.sync_copy(src_h.at[s, j], src_v.at[b])
                pltpu.sync_copy(dst_h.at[s, j], dst_v.at[b])

            def fire(b):
                pltpu.async_copy(tab.at[src_v.at[b]], rows_v.at[b], sems[b])

            def drain(b):
                pltpu.make_async_copy(tab.at[src_v.at[b]], rows_v.at[b],
                                      sems[b]).wait()

            def sadd(b):
                pltpu.sync_copy(rows_v.at[b], acc.at[dst_v.at[b]], add=True)

            fetch_idx(0, 0)
            fire(0)

            def pair(i, carry):
                w0 = 2 * i
                fetch_idx(w0 + 1, 1)
                drain(0)
                fire(1)
                sadd(0)

                @pl.when(i < _WPW // 2 - 1)
                def _():
                    fetch_idx(w0 + 2, 0)
                    drain(1)
                    fire(0)
                    sadd(1)

                @pl.when(i == _WPW // 2 - 1)
                def _():
                    drain(1)
                    sadd(1)

                return carry

            lax.fori_loop(0, _WPW // 2, pair, 0)
            plsc.subcore_barrier()

            @pl.when(s < _NWORK - 1)
            def _():
                pltpu.sync_copy(acc.at[pl.ds(s * _FCH, _FCH)],
                                out.at[pl.ds(s * _FCH, _FCH)])

            @pl.when(s == _NWORK - 1)
            def _():
                pltpu.sync_copy(acc.at[pl.ds(s * _FCH, _FLAST)],
                                out.at[pl.ds(s * _FCH, _FLAST)])

            plsc.subcore_barrier()

        @pl.when(c == 0)
        def _():
            run_quarter(h0, o0)
            run_quarter(h1, o1)

        @pl.when(c == 1)
        def _():
            run_quarter(h2, o2)
            run_quarter(h3, o3)

    return k(hq[0], hq[1], hq[2], hq[3], src3, dst3)


def _seg_mean(hq, src3, dst3, inv):
    q = _seg_sum_sc(hq, src3, dst3)
    s = jnp.concatenate(q, axis=1)
    return s * inv[:, None]


def kernel(x_encounter, x_patient, edge_index_enc_to_pat, edge_index_pat_to_enc,
           W_enc, b_enc, emb_pat, W_rel, W_root, b_conv, W_cls, b_cls):
    f32 = jnp.float32
    src0 = edge_index_enc_to_pat[0]
    dst0 = edge_index_enc_to_pat[1]
    src1 = edge_index_pat_to_enc[0]
    dst1 = edge_index_pat_to_enc[1]
    s0_3, d0_3 = _pad_edges(src0, dst0)
    s1_3, d1_3 = _pad_edges(src1, dst1)

    ones = jnp.ones((_E,), f32)
    c0 = jax.ops.segment_sum(ones, dst0, num_segments=_NP)
    c1 = jax.ops.segment_sum(ones, dst1, num_segments=_NE)
    inv0 = 1.0 / jnp.maximum(c0, 1.0)
    inv1 = 1.0 / jnp.maximum(c1, 1.0)

    enc = _mm_bias(x_encounter, W_enc, b_enc)
    pat = jnp.take(emb_pat, x_patient, axis=0)

    A_enc = _mm_bias(enc, W_root[0], b_conv[0])
    A_pat = _mm_bias(pat, W_root[0], b_conv[0])
    h0q = _mm_slices(enc, W_rel[0, 0])
    h1q = _mm_slices(pat, W_rel[0, 1])
    s0 = _seg_mean(h0q, s0_3, d0_3, inv0)
    s1 = _seg_mean(h1q, s1_3, d1_3, inv1)
    enc = jax.nn.relu(A_enc + s1)
    pat = jax.nn.relu(A_pat + s0)

    A_enc = _mm_bias(enc, W_root[1], b_conv[1])
    h1q = _mm_slices(pat, W_rel[1, 1])
    s1 = _seg_mean(h1q, s1_3, d1_3, inv1)
    enc = jax.nn.relu(A_enc + s1)

    logit = (enc @ W_cls + b_cls).reshape(-1)
    return logit


# speedup vs baseline: 4.6539x; 1.3214x over previous
import functools

import jax
import jax.numpy as jnp
from jax import lax
from jax.experimental import pallas as pl
from jax.experimental.pallas import tpu as pltpu
from jax.experimental.pallas import tpu_sc as plsc

_NE = 50000
_NP = 50000
_E = 300000
_H = 128

_NWORK = 16
_WIN = 128
_WPW = 148
_EPW = _WIN * _WPW
_EPAD = _EPW * _NWORK
_SENT = 176
_NROWS = _NP + _SENT
_ZCH = _NROWS // _NWORK
_FCH = _ZCH
_FLAST = _NP - 15 * _FCH


_PPW = 1664
_PWIN = _PPW // _WIN
_PPAD = _PPW * 32


def _mm_bias(x, W, b, block=1000, rows=None):
    N, H = x.shape
    Ho = W.shape[1]
    b2 = b.reshape(1, Ho)
    N = rows if rows is not None else N

    def body(x_ref, w_ref, b_ref, o_ref):
        o_ref[...] = jnp.dot(x_ref[...], w_ref[...],
                             preferred_element_type=jnp.float32) + b_ref[...]

    return pl.pallas_call(
        body,
        grid=(N // block,),
        in_specs=[pl.BlockSpec((block, H), lambda i: (i, 0)),
                  pl.BlockSpec((H, Ho), lambda i: (0, 0)),
                  pl.BlockSpec((1, Ho), lambda i: (0, 0))],
        out_specs=pl.BlockSpec((block, Ho), lambda i: (i, 0)),
        out_shape=jax.ShapeDtypeStruct((N, Ho), jnp.float32),
    )(x, W, b2)


def _mm_slices(x, W, block=1000, rows=None):
    N, H = x.shape
    N = rows if rows is not None else N

    def body(x_ref, w_ref, *outs):
        r = jnp.dot(x_ref[...], w_ref[...], preferred_element_type=jnp.float32)
        for q, o in enumerate(outs):
            o[...] = r[:, q * 32:(q + 1) * 32]

    return pl.pallas_call(
        body,
        grid=(N // block,),
        in_specs=[pl.BlockSpec((block, H), lambda i: (i, 0)),
                  pl.BlockSpec((H, H), lambda i: (0, 0))],
        out_specs=[pl.BlockSpec((block, 32), lambda i: (i, 0))] * 4,
        out_shape=[jax.ShapeDtypeStruct((N, 32), jnp.float32)] * 4,
    )(x, W)


def _counts_emb_sc(dst0_3, dst1_3, emb, xpat3):
    mesh = plsc.VectorSubcoreMesh(core_axis_name="c", subcore_axis_name="s")

    @functools.partial(
        pl.kernel, mesh=mesh,
        compiler_params=pltpu.CompilerParams(use_tc_tiling_on_sc=False),
        out_type=[jax.ShapeDtypeStruct((_NP,), jnp.float32)] * 2
        + [jax.ShapeDtypeStruct((_PPAD, 128), jnp.float32)],
        scratch_types=[
            pltpu.VMEM((2, _WIN), jnp.int32),
            pltpu.VMEM((_WIN,), jnp.float32),
            pltpu.VMEM((3136,), jnp.float32),
            pltpu.VMEM((2, _WIN, 128), jnp.float32),
            pltpu.VMEM_SHARED((_NROWS,), jnp.float32),
            pltpu.SemaphoreType.DMA,
            pltpu.SemaphoreType.DMA,
        ],
    )
    def k(d0_h, d1_h, emb_h, xp_h, c0_o, c1_o, pat_o,
          iw, ones_v, zero_v, erows, acc, sem0, sem1):
        c = lax.axis_index("c")
        s = lax.axis_index("s")
        sems = (sem0, sem1)

        def fill(i, carry):
            ones_v[pl.ds(i * 16, 16)] = jnp.full((16,), 1.0, jnp.float32)
            return carry

        lax.fori_loop(0, _WIN // 16, fill, 0)

        def zfill(i, carry):
            zero_v[pl.ds(i * 16, 16)] = jnp.zeros((16,), jnp.float32)
            return carry

        lax.fori_loop(0, 3136 // 16, zfill, 0)

        def run_counts(d_h, out):
            pltpu.sync_copy(zero_v, acc.at[pl.ds(s * _ZCH, _ZCH)])
            plsc.subcore_barrier()

            def win(j, carry):
                pltpu.sync_copy(d_h.at[s, j], iw.at[0])
                pltpu.sync_copy(ones_v, acc.at[iw.at[0]], add=True)
                return carry

            lax.fori_loop(0, _WPW, win, 0)
            plsc.subcore_barrier()

            @pl.when(s < _NWORK - 1)
            def _():
                pltpu.sync_copy(acc.at[pl.ds(s * _FCH, _FCH)],
                                out.at[pl.ds(s * _FCH, _FCH)])

            @pl.when(s == _NWORK - 1)
            def _():
                pltpu.sync_copy(acc.at[pl.ds(s * _FCH, _FLAST)],
                                out.at[pl.ds(s * _FCH, _FLAST)])

        @pl.when(c == 0)
        def _():
            run_counts(d0_h, c0_o)

        @pl.when(c == 1)
        def _():
            run_counts(d1_h, c1_o)

        wid = c * _NWORK + s
        base = wid * _PPW

        def efetch(j, b):
            pltpu.sync_copy(xp_h.at[wid, j], iw.at[b])

        def efire(b):
            pltpu.async_copy(emb_h.at[iw.at[b]], erows.at[b], sems[b])

        def edrain(b):
            pltpu.make_async_copy(emb_h.at[iw.at[b]], erows.at[b],
                                  sems[b]).wait()

        def eout(j, b):
            pltpu.sync_copy(erows.at[b],
                            pat_o.at[pl.ds(base + j * _WIN, _WIN)])

        efetch(0, 0)
        efire(0)

        def epair(i, carry):
            w0 = 2 * i
            efetch(w0 + 1, 1)
            edrain(0)
            efire(1)
            eout(w0, 0)
            efetch(w0 + 2, 0)
            edrain(1)
            efire(0)
            eout(w0 + 1, 1)
            return carry

        lax.fori_loop(0, (_PWIN - 1) // 2, epair, 0)
        edrain(0)
        eout(_PWIN - 1, 0)

    return k(dst0_3, dst1_3, emb, xpat3)


def _pad_edges(src, dst):
    pad = _EPAD - _E
    i = jnp.arange(pad, dtype=jnp.int32)
    src_p = jnp.concatenate([src.astype(jnp.int32), i % _NE])
    dst_p = jnp.concatenate([dst.astype(jnp.int32), _NP + (i % _SENT)])
    return (src_p.reshape(_NWORK, _WPW, _WIN),
            dst_p.reshape(_NWORK, _WPW, _WIN))


def _seg_sum_sc(hq, src3, dst3):
    mesh = plsc.VectorSubcoreMesh(core_axis_name="c", subcore_axis_name="s")

    @functools.partial(
        pl.kernel, mesh=mesh,
        compiler_params=pltpu.CompilerParams(use_tc_tiling_on_sc=False),
        out_type=[jax.ShapeDtypeStruct((_NP, 32), jnp.float32)] * 4,
        scratch_types=[
            pltpu.VMEM((2, _WIN), jnp.int32),
            pltpu.VMEM((2, _WIN), jnp.int32),
            pltpu.VMEM((2, _WIN, 32), jnp.float32),
            pltpu.VMEM((196, 32), jnp.float32),
            pltpu.VMEM_SHARED((_NROWS, 32), jnp.float32),
            pltpu.SemaphoreType.DMA,
            pltpu.SemaphoreType.DMA,
        ],
    )
    def k(h0, h1, h2, h3, src_h, dst_h, o0, o1, o2, o3,
          src_v, dst_v, rows_v, zero_v, acc, sem0, sem1):
        c = lax.axis_index("c")
        s = lax.axis_index("s")

        def zrow(i, carry):
            zero_v[i, 0:16] = jnp.zeros((16,), jnp.float32)
            zero_v[i, 16:32] = jnp.zeros((16,), jnp.float32)
            return carry

        lax.fori_loop(0, 196, zrow, 0)
        sems = (sem0, sem1)

        def run_quarter(tab, out):
            def zchunk(i, carry):
                pltpu.sync_copy(zero_v,
                                acc.at[pl.ds(s * _ZCH + i * 196, 196)])
                return carry

            lax.fori_loop(0, _ZCH // 196, zchunk, 0)
            plsc.subcore_barrier()

            def fetch_idx(j, b):
                pltpu.sync_copy(src_h.at[s, j], src_v.at[b])
                pltpu.sync_copy(dst_h.at[s, j], dst_v.at[b])

            def fire(b):
                pltpu.async_copy(tab.at[src_v.at[b]], rows_v.at[b], sems[b])

            def drain(b):
                pltpu.make_async_copy(tab.at[src_v.at[b]], rows_v.at[b],
                                      sems[b]).wait()

            def sadd(b):
                pltpu.sync_copy(rows_v.at[b], acc.at[dst_v.at[b]], add=True)

            fetch_idx(0, 0)
            fire(0)

            def pair(i, carry):
                w0 = 2 * i
                fetch_idx(w0 + 1, 1)
                drain(0)
                fire(1)
                sadd(0)

                @pl.when(i < _WPW // 2 - 1)
                def _():
                    fetch_idx(w0 + 2, 0)
                    drain(1)
                    fire(0)
                    sadd(1)

                @pl.when(i == _WPW // 2 - 1)
                def _():
                    drain(1)
                    sadd(1)

                return carry

            lax.fori_loop(0, _WPW // 2, pair, 0)
            plsc.subcore_barrier()

            @pl.when(s < _NWORK - 1)
            def _():
                pltpu.sync_copy(acc.at[pl.ds(s * _FCH, _FCH)],
                                out.at[pl.ds(s * _FCH, _FCH)])

            @pl.when(s == _NWORK - 1)
            def _():
                pltpu.sync_copy(acc.at[pl.ds(s * _FCH, _FLAST)],
                                out.at[pl.ds(s * _FCH, _FLAST)])

            plsc.subcore_barrier()

        @pl.when(c == 0)
        def _():
            run_quarter(h0, o0)
            run_quarter(h1, o1)

        @pl.when(c == 1)
        def _():
            run_quarter(h2, o2)
            run_quarter(h3, o3)

    return k(hq[0], hq[1], hq[2], hq[3], src3, dst3)


def _seg_mean(hq, src3, dst3, inv):
    q = _seg_sum_sc(hq, src3, dst3)
    s = jnp.concatenate(q, axis=1)
    return s * inv[:, None]


def kernel(x_encounter, x_patient, edge_index_enc_to_pat, edge_index_pat_to_enc,
           W_enc, b_enc, emb_pat, W_rel, W_root, b_conv, W_cls, b_cls):
    f32 = jnp.float32
    src0 = edge_index_enc_to_pat[0]
    dst0 = edge_index_enc_to_pat[1]
    src1 = edge_index_pat_to_enc[0]
    dst1 = edge_index_pat_to_enc[1]
    s0_3, d0_3 = _pad_edges(src0, dst0)
    s1_3, d1_3 = _pad_edges(src1, dst1)
    ppad = jnp.arange(_PPAD - _NP, dtype=jnp.int32) % 10000
    xpat3 = jnp.concatenate([x_patient.astype(jnp.int32),
                             ppad]).reshape(32, _PWIN, _WIN)

    c0, c1, pat = _counts_emb_sc(d0_3, d1_3, emb_pat, xpat3)
    inv0 = 1.0 / jnp.maximum(c0, 1.0)
    inv1 = 1.0 / jnp.maximum(c1, 1.0)

    enc = _mm_bias(x_encounter, W_enc, b_enc)

    A_enc = _mm_bias(enc, W_root[0], b_conv[0])
    A_pat = _mm_bias(pat, W_root[0], b_conv[0], rows=_NP)
    h0q = _mm_slices(enc, W_rel[0, 0])
    h1q = _mm_slices(pat, W_rel[0, 1], rows=_NP)
    s0 = _seg_mean(h0q, s0_3, d0_3, inv0)
    s1 = _seg_mean(h1q, s1_3, d1_3, inv1)
    enc = jax.nn.relu(A_enc + s1)
    pat = jax.nn.relu(A_pat + s0)

    A_enc = _mm_bias(enc, W_root[1], b_conv[1])
    h1q = _mm_slices(pat, W_rel[1, 1])
    s1 = _seg_mean(h1q, s1_3, d1_3, inv1)
    enc = jax.nn.relu(A_enc + s1)

    logit = (enc @ W_cls + b_cls).reshape(-1)
    return logit


# speedup vs baseline: 6.3133x; 1.3566x over previous
import functools

import jax
import jax.numpy as jnp
from jax import lax
from jax.experimental import pallas as pl
from jax.experimental.pallas import tpu as pltpu
from jax.experimental.pallas import tpu_sc as plsc

_NE = 50000
_NP = 50000
_E = 300000
_H = 128

_NWORK = 16
_WIN = 128
_WPW = 150
_GRP = 6
_EPW = _WIN * _WPW
_EPAD = _EPW * _NWORK
_SENT = 176
_NROWS = _NP + _SENT
_ZCH = _NROWS // _NWORK
_FCH = _ZCH
_FLAST = _NP - 15 * _FCH


_PPW = 1664
_PWIN = _PPW // _WIN
_PPAD = _PPW * 32


def _mm_bias(x, W, b, block=1000, rows=None):
    N, H = x.shape
    Ho = W.shape[1]
    b2 = b.reshape(1, Ho)
    N = rows if rows is not None else N

    def body(x_ref, w_ref, b_ref, o_ref):
        o_ref[...] = jnp.dot(x_ref[...], w_ref[...],
                             preferred_element_type=jnp.float32) + b_ref[...]

    return pl.pallas_call(
        body,
        grid=(N // block,),
        in_specs=[pl.BlockSpec((block, H), lambda i: (i, 0)),
                  pl.BlockSpec((H, Ho), lambda i: (0, 0)),
                  pl.BlockSpec((1, Ho), lambda i: (0, 0))],
        out_specs=pl.BlockSpec((block, Ho), lambda i: (i, 0)),
        out_shape=jax.ShapeDtypeStruct((N, Ho), jnp.float32),
    )(x, W, b2)


def _mm_slices(x, W, block=1000, rows=None):
    N, H = x.shape
    N = rows if rows is not None else N

    def body(x_ref, w_ref, *outs):
        r = jnp.dot(x_ref[...], w_ref[...], preferred_element_type=jnp.float32)
        for q, o in enumerate(outs):
            o[...] = r[:, q * 32:(q + 1) * 32]

    return pl.pallas_call(
        body,
        grid=(N // block,),
        in_specs=[pl.BlockSpec((block, H), lambda i: (i, 0)),
                  pl.BlockSpec((H, H), lambda i: (0, 0))],
        out_specs=[pl.BlockSpec((block, 32), lambda i: (i, 0))] * 4,
        out_shape=[jax.ShapeDtypeStruct((N, 32), jnp.float32)] * 4,
    )(x, W)


def _counts_emb_sc(dst0_3, dst1_3, emb, xpat3):
    mesh = plsc.VectorSubcoreMesh(core_axis_name="c", subcore_axis_name="s")

    @functools.partial(
        pl.kernel, mesh=mesh,
        compiler_params=pltpu.CompilerParams(use_tc_tiling_on_sc=False),
        out_type=[jax.ShapeDtypeStruct((_NP,), jnp.float32)] * 2
        + [jax.ShapeDtypeStruct((_PPAD, 128), jnp.float32)],
        scratch_types=[
            pltpu.VMEM((2, _WIN), jnp.int32),
            pltpu.VMEM((_WIN,), jnp.float32),
            pltpu.VMEM((3136,), jnp.float32),
            pltpu.VMEM((2, _WIN, 128), jnp.float32),
            pltpu.VMEM_SHARED((_NROWS,), jnp.float32),
            pltpu.SemaphoreType.DMA,
            pltpu.SemaphoreType.DMA,
        ],
    )
    def k(d0_h, d1_h, emb_h, xp_h, c0_o, c1_o, pat_o,
          iw, ones_v, zero_v, erows, acc, sem0, sem1):
        c = lax.axis_index("c")
        s = lax.axis_index("s")
        sems = (sem0, sem1)

        def fill(i, carry):
            ones_v[pl.ds(i * 16, 16)] = jnp.full((16,), 1.0, jnp.float32)
            return carry

        lax.fori_loop(0, _WIN // 16, fill, 0)

        def zfill(i, carry):
            zero_v[pl.ds(i * 16, 16)] = jnp.zeros((16,), jnp.float32)
            return carry

        lax.fori_loop(0, 3136 // 16, zfill, 0)

        def run_counts(d_h, out):
            pltpu.sync_copy(zero_v, acc.at[pl.ds(s * _ZCH, _ZCH)])
            plsc.subcore_barrier()

            def win(j, carry):
                pltpu.sync_copy(d_h.at[s, j], iw.at[0])
                pltpu.sync_copy(ones_v, acc.at[iw.at[0]], add=True)
                return carry

            lax.fori_loop(0, _WPW, win, 0)
            plsc.subcore_barrier()

            @pl.when(s < _NWORK - 1)
            def _():
                pltpu.sync_copy(acc.at[pl.ds(s * _FCH, _FCH)],
                                out.at[pl.ds(s * _FCH, _FCH)])

            @pl.when(s == _NWORK - 1)
            def _():
                pltpu.sync_copy(acc.at[pl.ds(s * _FCH, _FLAST)],
                                out.at[pl.ds(s * _FCH, _FLAST)])

        @pl.when(c == 0)
        def _():
            run_counts(d0_h, c0_o)

        @pl.when(c == 1)
        def _():
            run_counts(d1_h, c1_o)

        wid = c * _NWORK + s
        base = wid * _PPW

        def efetch(j, b):
            pltpu.sync_copy(xp_h.at[wid, j], iw.at[b])

        def efire(b):
            pltpu.async_copy(emb_h.at[iw.at[b]], erows.at[b], sems[b])

        def edrain(b):
            pltpu.make_async_copy(emb_h.at[iw.at[b]], erows.at[b],
                                  sems[b]).wait()

        def eout(j, b):
            pltpu.sync_copy(erows.at[b],
                            pat_o.at[pl.ds(base + j * _WIN, _WIN)])

        efetch(0, 0)
        efire(0)

        def epair(i, carry):
            w0 = 2 * i
            efetch(w0 + 1, 1)
            edrain(0)
            efire(1)
            eout(w0, 0)
            efetch(w0 + 2, 0)
            edrain(1)
            efire(0)
            eout(w0 + 1, 1)
            return carry

        lax.fori_loop(0, (_PWIN - 1) // 2, epair, 0)
        edrain(0)
        eout(_PWIN - 1, 0)

    return k(dst0_3, dst1_3, emb, xpat3)


def _pad_edges(src, dst):
    pad = _EPAD - _E
    i = jnp.arange(pad, dtype=jnp.int32)
    src_p = jnp.concatenate([src.astype(jnp.int32), i % _NE])
    dst_p = jnp.concatenate([dst.astype(jnp.int32), _NP + (i % _SENT)])
    return (src_p.reshape(_NWORK, _WPW, _WIN),
            dst_p.reshape(_NWORK, _WPW, _WIN))


def _seg_sum_sc(hq, src3, dst3):
    mesh = plsc.VectorSubcoreMesh(core_axis_name="c", subcore_axis_name="s")

    @functools.partial(
        pl.kernel, mesh=mesh,
        compiler_params=pltpu.CompilerParams(use_tc_tiling_on_sc=False),
        out_type=[jax.ShapeDtypeStruct((_NP, 32), jnp.float32)] * 4,
        scratch_types=[
            pltpu.VMEM((_GRP, _WIN), jnp.int32),
            pltpu.VMEM((_GRP, _WIN), jnp.int32),
            pltpu.VMEM((3, _WIN, 32), jnp.float32),
            pltpu.VMEM((196, 32), jnp.float32),
            pltpu.VMEM_SHARED((_NROWS, 32), jnp.float32),
            [pltpu.SemaphoreType.DMA] * 3,
            [pltpu.SemaphoreType.DMA] * 3,
        ],
    )
    def k(h0, h1, h2, h3, src_h, dst_h, o0, o1, o2, o3,
          src_v, dst_v, rows_v, zero_v, acc, semg, sems):
        c = lax.axis_index("c")
        s = lax.axis_index("s")

        def zrow(i, carry):
            zero_v[i, 0:16] = jnp.zeros((16,), jnp.float32)
            zero_v[i, 16:32] = jnp.zeros((16,), jnp.float32)
            return carry

        lax.fori_loop(0, 196, zrow, 0)

        def run_quarter(tab, out):
            def zchunk(i, carry):
                pltpu.sync_copy(zero_v,
                                acc.at[pl.ds(s * _ZCH + i * 196, 196)])
                return carry

            lax.fori_loop(0, _ZCH // 196, zchunk, 0)
            plsc.subcore_barrier()

            def fire_g(w, b):
                pltpu.async_copy(tab.at[src_v.at[w]], rows_v.at[b], semg[b])

            def drain_g(b):
                pltpu.make_async_copy(tab.at[src_v.at[0]], rows_v.at[b],
                                      semg[b]).wait()

            def fire_s(w, b):
                pltpu.async_copy(rows_v.at[b], acc.at[dst_v.at[w]],
                                 sems[b], add=True)

            def drain_s(b):
                pltpu.make_async_copy(rows_v.at[b], acc.at[dst_v.at[0]],
                                      sems[b]).wait()

            def group(g, carry):
                @pl.when(g > 0)
                def _():
                    for b in range(3):
                        drain_s(b)

                pltpu.sync_copy(src_h.at[s, pl.ds(g * _GRP, _GRP)], src_v)
                pltpu.sync_copy(dst_h.at[s, pl.ds(g * _GRP, _GRP)], dst_v)
                for b in range(3):
                    fire_g(b, b)
                for b in range(3):
                    drain_g(b)
                    fire_s(b, b)
                for b in range(3):
                    drain_s(b)
                    fire_g(3 + b, b)
                for b in range(3):
                    drain_g(b)
                    fire_s(3 + b, b)
                return carry

            lax.fori_loop(0, _WPW // _GRP, group, 0)
            for b in range(3):
                drain_s(b)
            plsc.subcore_barrier()

            @pl.when(s < _NWORK - 1)
            def _():
                pltpu.sync_copy(acc.at[pl.ds(s * _FCH, _FCH)],
                                out.at[pl.ds(s * _FCH, _FCH)])

            @pl.when(s == _NWORK - 1)
            def _():
                pltpu.sync_copy(acc.at[pl.ds(s * _FCH, _FLAST)],
                                out.at[pl.ds(s * _FCH, _FLAST)])

            plsc.subcore_barrier()

        @pl.when(c == 0)
        def _():
            run_quarter(h0, o0)
            run_quarter(h1, o1)

        @pl.when(c == 1)
        def _():
            run_quarter(h2, o2)
            run_quarter(h3, o3)

    return k(hq[0], hq[1], hq[2], hq[3], src3, dst3)


def _seg_mean(hq, src3, dst3, inv):
    q = _seg_sum_sc(hq, src3, dst3)
    s = jnp.concatenate(q, axis=1)
    return s * inv[:, None]


def kernel(x_encounter, x_patient, edge_index_enc_to_pat, edge_index_pat_to_enc,
           W_enc, b_enc, emb_pat, W_rel, W_root, b_conv, W_cls, b_cls):
    f32 = jnp.float32
    src0 = edge_index_enc_to_pat[0]
    dst0 = edge_index_enc_to_pat[1]
    src1 = edge_index_pat_to_enc[0]
    dst1 = edge_index_pat_to_enc[1]
    s0_3, d0_3 = _pad_edges(src0, dst0)
    s1_3, d1_3 = _pad_edges(src1, dst1)
    ppad = jnp.arange(_PPAD - _NP, dtype=jnp.int32) % 10000
    xpat3 = jnp.concatenate([x_patient.astype(jnp.int32),
                             ppad]).reshape(32, _PWIN, _WIN)

    c0, c1, pat = _counts_emb_sc(d0_3, d1_3, emb_pat, xpat3)
    inv0 = 1.0 / jnp.maximum(c0, 1.0)
    inv1 = 1.0 / jnp.maximum(c1, 1.0)

    enc = _mm_bias(x_encounter, W_enc, b_enc)

    A_enc = _mm_bias(enc, W_root[0], b_conv[0])
    A_pat = _mm_bias(pat, W_root[0], b_conv[0], rows=_NP)
    h0q = _mm_slices(enc, W_rel[0, 0])
    h1q = _mm_slices(pat, W_rel[0, 1], rows=_NP)
    s0 = _seg_mean(h0q, s0_3, d0_3, inv0)
    s1 = _seg_mean(h1q, s1_3, d1_3, inv1)
    enc = jax.nn.relu(A_enc + s1)
    pat = jax.nn.relu(A_pat + s0)

    A_enc = _mm_bias(enc, W_root[1], b_conv[1])
    h1q = _mm_slices(pat, W_rel[1, 1])
    s1 = _seg_mean(h1q, s1_3, d1_3, inv1)
    enc = jax.nn.relu(A_enc + s1)

    logit = (enc @ W_cls + b_cls).reshape(-1)
    return logit


# speedup vs baseline: 6.4911x; 1.0282x over previous
import functools

import jax
import jax.numpy as jnp
from jax import lax
from jax.experimental import pallas as pl
from jax.experimental.pallas import tpu as pltpu
from jax.experimental.pallas import tpu_sc as plsc

_NE = 50000
_NP = 50000
_E = 300000
_H = 128

_NWORK = 16
_WIN = 128
_WPW = 150
_GRP = 6
_EPW = _WIN * _WPW
_EPAD = _EPW * _NWORK
_SENT = 176
_NROWS = _NP + _SENT
_ZCH = _NROWS // _NWORK
_FCH = _ZCH
_FLAST = _NP - 15 * _FCH


_PPW = 1664
_PWIN = _PPW // _WIN
_PPAD = _PPW * 32


_BLK = 1000
_CONST = lambda i: (0, 0)
_ROWB = lambda i: (i, 0)


def _qspecs():
    return [pl.BlockSpec((_BLK, 32), _ROWB)] * 4


def _qshapes(N):
    return [jax.ShapeDtypeStruct((N, 32), jnp.float32)] * 4


def _layer0_tc(x, We, be, Wr, br, Wq, rows=None):
    N = rows if rows is not None else x.shape[0]
    proj = We is not None

    def body(*refs):
        if proj:
            x_ref, we, be_r, wr, br_r, wq = refs[:6]
            e = jnp.dot(x_ref[...], we[...],
                        preferred_element_type=jnp.float32) + be_r[...]
        else:
            x_ref, wr, br_r, wq = refs[:4]
            e = x_ref[...]
        oA = refs[-5]
        outs = refs[-4:]
        oA[...] = jnp.dot(e, wr[...],
                          preferred_element_type=jnp.float32) + br_r[...]
        r = jnp.dot(e, wq[...], preferred_element_type=jnp.float32)
        for q, o in enumerate(outs):
            o[...] = r[:, q * 32:(q + 1) * 32]

    xspec = pl.BlockSpec((_BLK, 128), _ROWB)
    wspec = pl.BlockSpec((128, 128), _CONST)
    bspec = pl.BlockSpec((1, 128), _CONST)
    in_specs = ([xspec, wspec, bspec, wspec, bspec, wspec] if proj
                else [xspec, wspec, bspec, wspec])
    args = ((x, We, be.reshape(1, 128), Wr, br.reshape(1, 128), Wq) if proj
            else (x, Wr, br.reshape(1, 128), Wq))
    out = pl.pallas_call(
        body,
        grid=(N // _BLK,),
        in_specs=in_specs,
        out_specs=[pl.BlockSpec((_BLK, 128), _ROWB)] + _qspecs(),
        out_shape=[jax.ShapeDtypeStruct((N, 128), jnp.float32)] + _qshapes(N),
    )(*args)
    return out[0], out[1:]


def _fused_next_tc(A, qs, inv, W, b, mode):
    N = A.shape[0]
    inv2 = inv.reshape(N, 1)
    Ho = W.shape[1]

    def body(a_ref, q0, q1, q2, q3, i_ref, w_ref, *rest):
        s = jnp.concatenate([q0[...], q1[...], q2[...], q3[...]], axis=1)
        x = jax.nn.relu(a_ref[...] + s * i_ref[...])
        r = jnp.dot(x, w_ref[...], preferred_element_type=jnp.float32)
        if mode == 'quarters':
            for q, o in enumerate(rest):
                o[...] = r[:, q * 32:(q + 1) * 32]
        else:
            b_ref, o_ref = rest
            o_ref[...] = r + b_ref[...]

    in_specs = [pl.BlockSpec((_BLK, 128), _ROWB)] + _qspecs() + [
        pl.BlockSpec((_BLK, 1), _ROWB),
        pl.BlockSpec((128, Ho), _CONST)]
    args = [A, *qs, inv2, W]
    if mode == 'quarters':
        out_specs, out_shape = _qspecs(), _qshapes(N)
    else:
        in_specs.append(pl.BlockSpec((1, Ho), _CONST))
        args.append(b.reshape(1, Ho))
        out_specs = pl.BlockSpec((_BLK, Ho), _ROWB)
        out_shape = jax.ShapeDtypeStruct((N, Ho), jnp.float32)

    return pl.pallas_call(
        body,
        grid=(N // _BLK,),
        in_specs=in_specs,
        out_specs=out_specs,
        out_shape=out_shape,
    )(*args)


def _counts_emb_sc(dst0_3, dst1_3, emb, xpat3):
    mesh = plsc.VectorSubcoreMesh(core_axis_name="c", subcore_axis_name="s")

    @functools.partial(
        pl.kernel, mesh=mesh,
        compiler_params=pltpu.CompilerParams(use_tc_tiling_on_sc=False),
        out_type=[jax.ShapeDtypeStruct((_NP,), jnp.float32)] * 2
        + [jax.ShapeDtypeStruct((_PPAD, 128), jnp.float32)],
        scratch_types=[
            pltpu.VMEM((2, _WIN), jnp.int32),
            pltpu.VMEM((_WIN,), jnp.float32),
            pltpu.VMEM((3136,), jnp.float32),
            pltpu.VMEM((2, _WIN, 128), jnp.float32),
            pltpu.VMEM_SHARED((_NROWS,), jnp.float32),
            pltpu.SemaphoreType.DMA,
            pltpu.SemaphoreType.DMA,
        ],
    )
    def k(d0_h, d1_h, emb_h, xp_h, c0_o, c1_o, pat_o,
          iw, ones_v, zero_v, erows, acc, sem0, sem1):
        c = lax.axis_index("c")
        s = lax.axis_index("s")
        sems = (sem0, sem1)

        def fill(i, carry):
            ones_v[pl.ds(i * 16, 16)] = jnp.full((16,), 1.0, jnp.float32)
            return carry

        lax.fori_loop(0, _WIN // 16, fill, 0)

        def zfill(i, carry):
            zero_v[pl.ds(i * 16, 16)] = jnp.zeros((16,), jnp.float32)
            return carry

        lax.fori_loop(0, 3136 // 16, zfill, 0)

        def run_counts(d_h, out):
            pltpu.sync_copy(zero_v, acc.at[pl.ds(s * _ZCH, _ZCH)])
            plsc.subcore_barrier()

            def win(j, carry):
                pltpu.sync_copy(d_h.at[s, j], iw.at[0])
                pltpu.sync_copy(ones_v, acc.at[iw.at[0]], add=True)
                return carry

            lax.fori_loop(0, _WPW, win, 0)
            plsc.subcore_barrier()

            @pl.when(s < _NWORK - 1)
            def _():
                pltpu.sync_copy(acc.at[pl.ds(s * _FCH, _FCH)],
                                out.at[pl.ds(s * _FCH, _FCH)])

            @pl.when(s == _NWORK - 1)
            def _():
                pltpu.sync_copy(acc.at[pl.ds(s * _FCH, _FLAST)],
                                out.at[pl.ds(s * _FCH, _FLAST)])

        @pl.when(c == 0)
        def _():
            run_counts(d0_h, c0_o)

        @pl.when(c == 1)
        def _():
            run_counts(d1_h, c1_o)

        wid = c * _NWORK + s
        base = wid * _PPW

        def efetch(j, b):
            pltpu.sync_copy(xp_h.at[wid, j], iw.at[b])

        def efire(b):
            pltpu.async_copy(emb_h.at[iw.at[b]], erows.at[b], sems[b])

        def edrain(b):
            pltpu.make_async_copy(emb_h.at[iw.at[b]], erows.at[b],
                                  sems[b]).wait()

        def eout(j, b):
            pltpu.sync_copy(erows.at[b],
                            pat_o.at[pl.ds(base + j * _WIN, _WIN)])

        efetch(0, 0)
        efire(0)

        def epair(i, carry):
            w0 = 2 * i
            efetch(w0 + 1, 1)
            edrain(0)
            efire(1)
            eout(w0, 0)
            efetch(w0 + 2, 0)
            edrain(1)
            efire(0)
            eout(w0 + 1, 1)
            return carry

        lax.fori_loop(0, (_PWIN - 1) // 2, epair, 0)
        edrain(0)
        eout(_PWIN - 1, 0)

    return k(dst0_3, dst1_3, emb, xpat3)


def _pad_edges(src, dst):
    pad = _EPAD - _E
    i = jnp.arange(pad, dtype=jnp.int32)
    src_p = jnp.concatenate([src.astype(jnp.int32), i % _NE])
    dst_p = jnp.concatenate([dst.astype(jnp.int32), _NP + (i % _SENT)])
    return (src_p.reshape(_NWORK, _WPW, _WIN),
            dst_p.reshape(_NWORK, _WPW, _WIN))


def _seg_sum_sc(hq, src3, dst3):
    mesh = plsc.VectorSubcoreMesh(core_axis_name="c", subcore_axis_name="s")

    @functools.partial(
        pl.kernel, mesh=mesh,
        compiler_params=pltpu.CompilerParams(use_tc_tiling_on_sc=False),
        out_type=[jax.ShapeDtypeStruct((_NP, 32), jnp.float32)] * 4,
        scratch_types=[
            pltpu.VMEM((_GRP, _WIN), jnp.int32),
            pltpu.VMEM((_GRP, _WIN), jnp.int32),
            pltpu.VMEM((3, _WIN, 32), jnp.float32),
            pltpu.VMEM((196, 32), jnp.float32),
            pltpu.VMEM_SHARED((_NROWS, 32), jnp.float32),
            [pltpu.SemaphoreType.DMA] * 3,
            [pltpu.SemaphoreType.DMA] * 3,
        ],
    )
    def k(h0, h1, h2, h3, src_h, dst_h, o0, o1, o2, o3,
          src_v, dst_v, rows_v, zero_v, acc, semg, sems):
        c = lax.axis_index("c")
        s = lax.axis_index("s")

        def zrow(i, carry):
            zero_v[i, 0:16] = jnp.zeros((16,), jnp.float32)
            zero_v[i, 16:32] = jnp.zeros((16,), jnp.float32)
            return carry

        lax.fori_loop(0, 196, zrow, 0)

        def run_quarter(tab, out):
            def zchunk(i, carry):
                pltpu.sync_copy(zero_v,
                                acc.at[pl.ds(s * _ZCH + i * 196, 196)])
                return carry

            lax.fori_loop(0, _ZCH // 196, zchunk, 0)
            plsc.subcore_barrier()

            def fire_g(w, b):
                pltpu.async_copy(tab.at[src_v.at[w]], rows_v.at[b], semg[b])

            def drain_g(b):
                pltpu.make_async_copy(tab.at[src_v.at[0]], rows_v.at[b],
                                      semg[b]).wait()

            def fire_s(w, b):
                pltpu.async_copy(rows_v.at[b], acc.at[dst_v.at[w]],
                                 sems[b], add=True)

            def drain_s(b):
                pltpu.make_async_copy(rows_v.at[b], acc.at[dst_v.at[0]],
                                      sems[b]).wait()

            def group(g, carry):
                @pl.when(g > 0)
                def _():
                    for b in range(3):
                        drain_s(b)

                pltpu.sync_copy(src_h.at[s, pl.ds(g * _GRP, _GRP)], src_v)
                pltpu.sync_copy(dst_h.at[s, pl.ds(g * _GRP, _GRP)], dst_v)
                for b in range(3):
                    fire_g(b, b)
                for b in range(3):
                    drain_g(b)
                    fire_s(b, b)
                for b in range(3):
                    drain_s(b)
                    fire_g(3 + b, b)
                for b in range(3):
                    drain_g(b)
                    fire_s(3 + b, b)
                return carry

            lax.fori_loop(0, _WPW // _GRP, group, 0)
            for b in range(3):
                drain_s(b)
            plsc.subcore_barrier()

            @pl.when(s < _NWORK - 1)
            def _():
                pltpu.sync_copy(acc.at[pl.ds(s * _FCH, _FCH)],
                                out.at[pl.ds(s * _FCH, _FCH)])

            @pl.when(s == _NWORK - 1)
            def _():
                pltpu.sync_copy(acc.at[pl.ds(s * _FCH, _FLAST)],
                                out.at[pl.ds(s * _FCH, _FLAST)])

            plsc.subcore_barrier()

        @pl.when(c == 0)
        def _():
            run_quarter(h0, o0)
            run_quarter(h1, o1)

        @pl.when(c == 1)
        def _():
            run_quarter(h2, o2)
            run_quarter(h3, o3)

    return k(hq[0], hq[1], hq[2], hq[3], src3, dst3)


def kernel(x_encounter, x_patient, edge_index_enc_to_pat, edge_index_pat_to_enc,
           W_enc, b_enc, emb_pat, W_rel, W_root, b_conv, W_cls, b_cls):
    f32 = jnp.float32
    src0 = edge_index_enc_to_pat[0]
    dst0 = edge_index_enc_to_pat[1]
    src1 = edge_index_pat_to_enc[0]
    dst1 = edge_index_pat_to_enc[1]
    s0_3, d0_3 = _pad_edges(src0, dst0)
    s1_3, d1_3 = _pad_edges(src1, dst1)
    ppad = jnp.arange(_PPAD - _NP, dtype=jnp.int32) % 10000
    xpat3 = jnp.concatenate([x_patient.astype(jnp.int32),
                             ppad]).reshape(32, _PWIN, _WIN)

    c0, c1, pat = _counts_emb_sc(d0_3, d1_3, emb_pat, xpat3)
    inv0 = 1.0 / jnp.maximum(c0, 1.0)
    inv1 = 1.0 / jnp.maximum(c1, 1.0)

    A_enc, h0q = _layer0_tc(x_encounter, W_enc, b_enc,
                            W_root[0], b_conv[0], W_rel[0, 0])
    A_pat, h1q = _layer0_tc(pat, None, None,
                            W_root[0], b_conv[0], W_rel[0, 1], rows=_NP)
    s0q = _seg_sum_sc(h0q, s0_3, d0_3)
    s1q = _seg_sum_sc(h1q, s1_3, d1_3)

    A_enc2 = _fused_next_tc(A_enc, s1q, inv1, W_root[1], b_conv[1], 'full')
    h1bq = _fused_next_tc(A_pat, s0q, inv0, W_rel[1, 1], None, 'quarters')
    s1bq = _seg_sum_sc(h1bq, s1_3, d1_3)

    logit = _fused_next_tc(A_enc2, s1bq, inv1, W_cls, b_cls, 'logit')
    return logit.reshape(-1)


# speedup vs baseline: 6.8369x; 1.0533x over previous
import functools

import jax
import jax.numpy as jnp
from jax import lax
from jax.experimental import pallas as pl
from jax.experimental.pallas import tpu as pltpu
from jax.experimental.pallas import tpu_sc as plsc

_NE = 50000
_NP = 50000
_E = 300000
_H = 128

_NWORK = 16
_WIN = 128
_WPW = 150
_GRP = 6
_EPW = _WIN * _WPW
_EPAD = _EPW * _NWORK
_SENT = 176
_NROWS = _NP + _SENT
_ZCH = _NROWS // _NWORK
_FCH = _ZCH
_FLAST = _NP - 15 * _FCH


_PPW = 1664
_PWIN = _PPW // _WIN
_PPAD = _PPW * 32


_BLK = 1000
_CONST = lambda i: (0, 0)
_ROWB = lambda i: (i, 0)


def _qspecs():
    return [pl.BlockSpec((_BLK, 32), _ROWB)] * 4


def _qshapes(N):
    return [jax.ShapeDtypeStruct((N, 32), jnp.bfloat16)] * 4


def _layer0_tc(x, We, be, Wr, br, Wq, rows=None):
    N = rows if rows is not None else x.shape[0]
    proj = We is not None

    def body(*refs):
        if proj:
            x_ref, we, be_r, wr, br_r, wq = refs[:6]
            e = jnp.dot(x_ref[...], we[...],
                        preferred_element_type=jnp.float32) + be_r[...]
        else:
            x_ref, wr, br_r, wq = refs[:4]
            e = x_ref[...]
        oA = refs[-5]
        outs = refs[-4:]
        oA[...] = jnp.dot(e, wr[...],
                          preferred_element_type=jnp.float32) + br_r[...]
        r = jnp.dot(e, wq[...], preferred_element_type=jnp.float32)
        for q, o in enumerate(outs):
            o[...] = r[:, q * 32:(q + 1) * 32].astype(jnp.bfloat16)

    xspec = pl.BlockSpec((_BLK, 128), _ROWB)
    wspec = pl.BlockSpec((128, 128), _CONST)
    bspec = pl.BlockSpec((1, 128), _CONST)
    in_specs = ([xspec, wspec, bspec, wspec, bspec, wspec] if proj
                else [xspec, wspec, bspec, wspec])
    args = ((x, We, be.reshape(1, 128), Wr, br.reshape(1, 128), Wq) if proj
            else (x, Wr, br.reshape(1, 128), Wq))
    out = pl.pallas_call(
        body,
        grid=(N // _BLK,),
        in_specs=in_specs,
        out_specs=[pl.BlockSpec((_BLK, 128), _ROWB)] + _qspecs(),
        out_shape=[jax.ShapeDtypeStruct((N, 128), jnp.float32)] + _qshapes(N),
    )(*args)
    return out[0], out[1:]


def _fused_next_tc(A, qs, inv, W, b, mode):
    N = A.shape[0]
    inv2 = inv.reshape(N, 1)
    Ho = W.shape[1]

    def body(a_ref, q0, q1, q2, q3, i_ref, w_ref, *rest):
        s = jnp.concatenate([q0[...], q1[...], q2[...], q3[...]],
                            axis=1).astype(jnp.float32)
        x = jax.nn.relu(a_ref[...] + s * i_ref[...])
        r = jnp.dot(x, w_ref[...], preferred_element_type=jnp.float32)
        if mode == 'quarters':
            for q, o in enumerate(rest):
                o[...] = r[:, q * 32:(q + 1) * 32].astype(jnp.bfloat16)
        else:
            b_ref, o_ref = rest
            o_ref[...] = r + b_ref[...]

    in_specs = [pl.BlockSpec((_BLK, 128), _ROWB)] + _qspecs() + [
        pl.BlockSpec((_BLK, 1), _ROWB),
        pl.BlockSpec((128, Ho), _CONST)]
    args = [A, *qs, inv2, W]
    if mode == 'quarters':
        out_specs, out_shape = _qspecs(), _qshapes(N)
    else:
        in_specs.append(pl.BlockSpec((1, Ho), _CONST))
        args.append(b.reshape(1, Ho))
        out_specs = pl.BlockSpec((_BLK, Ho), _ROWB)
        out_shape = jax.ShapeDtypeStruct((N, Ho), jnp.float32)

    return pl.pallas_call(
        body,
        grid=(N // _BLK,),
        in_specs=in_specs,
        out_specs=out_specs,
        out_shape=out_shape,
    )(*args)


def _counts_emb_sc(dst0_3, dst1_3, emb, xpat3):
    mesh = plsc.VectorSubcoreMesh(core_axis_name="c", subcore_axis_name="s")

    @functools.partial(
        pl.kernel, mesh=mesh,
        compiler_params=pltpu.CompilerParams(use_tc_tiling_on_sc=False),
        out_type=[jax.ShapeDtypeStruct((_NP,), jnp.float32)] * 2
        + [jax.ShapeDtypeStruct((_PPAD, 128), jnp.float32)],
        scratch_types=[
            pltpu.VMEM((2, _WIN), jnp.int32),
            pltpu.VMEM((_WIN,), jnp.float32),
            pltpu.VMEM((3136,), jnp.float32),
            pltpu.VMEM((2, _WIN, 128), jnp.float32),
            pltpu.VMEM_SHARED((_NROWS,), jnp.float32),
            pltpu.SemaphoreType.DMA,
            pltpu.SemaphoreType.DMA,
        ],
    )
    def k(d0_h, d1_h, emb_h, xp_h, c0_o, c1_o, pat_o,
          iw, ones_v, zero_v, erows, acc, sem0, sem1):
        c = lax.axis_index("c")
        s = lax.axis_index("s")
        sems = (sem0, sem1)

        def fill(i, carry):
            ones_v[pl.ds(i * 16, 16)] = jnp.full((16,), 1.0, jnp.float32)
            return carry

        lax.fori_loop(0, _WIN // 16, fill, 0)

        def zfill(i, carry):
            zero_v[pl.ds(i * 16, 16)] = jnp.zeros((16,), jnp.float32)
            return carry

        lax.fori_loop(0, 3136 // 16, zfill, 0)

        def run_counts(d_h, out):
            pltpu.sync_copy(zero_v, acc.at[pl.ds(s * _ZCH, _ZCH)])
            plsc.subcore_barrier()

            def win(j, carry):
                pltpu.sync_copy(d_h.at[s, j], iw.at[0])
                pltpu.sync_copy(ones_v, acc.at[iw.at[0]], add=True)
                return carry

            lax.fori_loop(0, _WPW, win, 0)
            plsc.subcore_barrier()

            @pl.when(s < _NWORK - 1)
            def _():
                pltpu.sync_copy(acc.at[pl.ds(s * _FCH, _FCH)],
                                out.at[pl.ds(s * _FCH, _FCH)])

            @pl.when(s == _NWORK - 1)
            def _():
                pltpu.sync_copy(acc.at[pl.ds(s * _FCH, _FLAST)],
                                out.at[pl.ds(s * _FCH, _FLAST)])

        @pl.when(c == 0)
        def _():
            run_counts(d0_h, c0_o)

        @pl.when(c == 1)
        def _():
            run_counts(d1_h, c1_o)

        wid = c * _NWORK + s
        base = wid * _PPW

        def efetch(j, b):
            pltpu.sync_copy(xp_h.at[wid, j], iw.at[b])

        def efire(b):
            pltpu.async_copy(emb_h.at[iw.at[b]], erows.at[b], sems[b])

        def edrain(b):
            pltpu.make_async_copy(emb_h.at[iw.at[b]], erows.at[b],
                                  sems[b]).wait()

        def eout(j, b):
            pltpu.sync_copy(erows.at[b],
                            pat_o.at[pl.ds(base + j * _WIN, _WIN)])

        efetch(0, 0)
        efire(0)

        def epair(i, carry):
            w0 = 2 * i
            efetch(w0 + 1, 1)
            edrain(0)
            efire(1)
            eout(w0, 0)
            efetch(w0 + 2, 0)
            edrain(1)
            efire(0)
            eout(w0 + 1, 1)
            return carry

        lax.fori_loop(0, (_PWIN - 1) // 2, epair, 0)
        edrain(0)
        eout(_PWIN - 1, 0)

    return k(dst0_3, dst1_3, emb, xpat3)


def _pad_edges(src, dst):
    pad = _EPAD - _E
    i = jnp.arange(pad, dtype=jnp.int32)
    src_p = jnp.concatenate([src.astype(jnp.int32), i % _NE])
    dst_p = jnp.concatenate([dst.astype(jnp.int32), _NP + (i % _SENT)])
    return (src_p.reshape(_NWORK, _WPW, _WIN),
            dst_p.reshape(_NWORK, _WPW, _WIN))


def _seg_sum_sc(hq, src3, dst3):
    mesh = plsc.VectorSubcoreMesh(core_axis_name="c", subcore_axis_name="s")

    @functools.partial(
        pl.kernel, mesh=mesh,
        compiler_params=pltpu.CompilerParams(use_tc_tiling_on_sc=False),
        out_type=[jax.ShapeDtypeStruct((_NP, 32), jnp.bfloat16)] * 4,
        scratch_types=[
            pltpu.VMEM((_GRP, _WIN), jnp.int32),
            pltpu.VMEM((_GRP, _WIN), jnp.int32),
            pltpu.VMEM((3, _WIN, 32), jnp.bfloat16),
            pltpu.VMEM((196, 32), jnp.bfloat16),
            pltpu.VMEM_SHARED((_NROWS, 32), jnp.bfloat16),
            [pltpu.SemaphoreType.DMA] * 3,
            [pltpu.SemaphoreType.DMA] * 3,
        ],
    )
    def k(h0, h1, h2, h3, src_h, dst_h, o0, o1, o2, o3,
          src_v, dst_v, rows_v, zero_v, acc, semg, sems):
        c = lax.axis_index("c")
        s = lax.axis_index("s")

        def zrow(i, carry):
            zero_v[i, :] = jnp.zeros((32,), jnp.bfloat16)
            return carry

        lax.fori_loop(0, 196, zrow, 0)

        def run_quarter(tab, out):
            def zchunk(i, carry):
                pltpu.sync_copy(zero_v,
                                acc.at[pl.ds(s * _ZCH + i * 196, 196)])
                return carry

            lax.fori_loop(0, _ZCH // 196, zchunk, 0)
            plsc.subcore_barrier()

            def fire_g(w, b):
                pltpu.async_copy(tab.at[src_v.at[w]], rows_v.at[b], semg[b])

            def drain_g(b):
                pltpu.make_async_copy(tab.at[src_v.at[0]], rows_v.at[b],
                                      semg[b]).wait()

            def fire_s(w, b):
                pltpu.async_copy(rows_v.at[b], acc.at[dst_v.at[w]],
                                 sems[b], add=True)

            def drain_s(b):
                pltpu.make_async_copy(rows_v.at[b], acc.at[dst_v.at[0]],
                                      sems[b]).wait()

            def group(g, carry):
                @pl.when(g > 0)
                def _():
                    for b in range(3):
                        drain_s(b)

                pltpu.sync_copy(src_h.at[s, pl.ds(g * _GRP, _GRP)], src_v)
                pltpu.sync_copy(dst_h.at[s, pl.ds(g * _GRP, _GRP)], dst_v)
                for b in range(3):
                    fire_g(b, b)
                for b in range(3):
                    drain_g(b)
                    fire_s(b, b)
                for b in range(3):
                    drain_s(b)
                    fire_g(3 + b, b)
                for b in range(3):
                    drain_g(b)
                    fire_s(3 + b, b)
                return carry

            lax.fori_loop(0, _WPW // _GRP, group, 0)
            for b in range(3):
                drain_s(b)
            plsc.subcore_barrier()

            @pl.when(s < _NWORK - 1)
            def _():
                pltpu.sync_copy(acc.at[pl.ds(s * _FCH, _FCH)],
                                out.at[pl.ds(s * _FCH, _FCH)])

            @pl.when(s == _NWORK - 1)
            def _():
                pltpu.sync_copy(acc.at[pl.ds(s * _FCH, _FLAST)],
                                out.at[pl.ds(s * _FCH, _FLAST)])

            plsc.subcore_barrier()

        @pl.when(c == 0)
        def _():
            run_quarter(h0, o0)
            run_quarter(h1, o1)

        @pl.when(c == 1)
        def _():
            run_quarter(h2, o2)
            run_quarter(h3, o3)

    return k(hq[0], hq[1], hq[2], hq[3], src3, dst3)


def kernel(x_encounter, x_patient, edge_index_enc_to_pat, edge_index_pat_to_enc,
           W_enc, b_enc, emb_pat, W_rel, W_root, b_conv, W_cls, b_cls):
    f32 = jnp.float32
    src0 = edge_index_enc_to_pat[0]
    dst0 = edge_index_enc_to_pat[1]
    src1 = edge_index_pat_to_enc[0]
    dst1 = edge_index_pat_to_enc[1]
    s0_3, d0_3 = _pad_edges(src0, dst0)
    s1_3, d1_3 = _pad_edges(src1, dst1)
    ppad = jnp.arange(_PPAD - _NP, dtype=jnp.int32) % 10000
    xpat3 = jnp.concatenate([x_patient.astype(jnp.int32),
                             ppad]).reshape(32, _PWIN, _WIN)

    c0, c1, pat = _counts_emb_sc(d0_3, d1_3, emb_pat, xpat3)
    inv0 = 1.0 / jnp.maximum(c0, 1.0)
    inv1 = 1.0 / jnp.maximum(c1, 1.0)

    A_enc, h0q = _layer0_tc(x_encounter, W_enc, b_enc,
                            W_root[0], b_conv[0], W_rel[0, 0])
    A_pat, h1q = _layer0_tc(pat, None, None,
                            W_root[0], b_conv[0], W_rel[0, 1], rows=_NP)
    s0q = _seg_sum_sc(h0q, s0_3, d0_3)
    s1q = _seg_sum_sc(h1q, s1_3, d1_3)

    A_enc2 = _fused_next_tc(A_enc, s1q, inv1, W_root[1], b_conv[1], 'full')
    h1bq = _fused_next_tc(A_pat, s0q, inv0, W_rel[1, 1], None, 'quarters')
    s1bq = _seg_sum_sc(h1bq, s1_3, d1_3)

    logit = _fused_next_tc(A_enc2, s1bq, inv1, W_cls, b_cls, 'logit')
    return logit.reshape(-1)
